# baseline (speedup 1.0000x reference)
import numpy as np

import jax
import jax.numpy as jnp
from jax.experimental import pallas as pl
from jax.experimental.pallas import tpu as pltpu

_LAYER_DIMS = [(784, 512), (512, 256), (256, 128), (128, 64), (64, 32), (32, 10)]
_CFG = [(4, 2), (2, 2), (2, 2), (2, 2), (2, 2), (2, 4)]

_BB = 2048


def _pad128(n):
    return ((n + 127) // 128) * 128


def _build_selectors():
    sin, bmask = [], []
    for li in range(6):
        din, dout = _LAYER_DIMS[li]
        ig, og = _CFG[li]
        p = ig * og
        grp_out = np.minimum(np.arange(dout) // (dout // og), og - 1)
        bm = np.zeros((p, dout), np.float32)
        for pi in range(p):
            bm[pi] = (grp_out == (pi % og)).astype(np.float32)
        bmask.append(bm)
        if li < 5:
            dpad = _pad128(din)
            grp_in = np.arange(din) // (din // ig)
            s = np.zeros((p, og * dpad), np.float32)
            for j in range(og):
                s[grp_in * og + j, j * dpad + np.arange(din)] = 1.0
            sin.append(s)
    din5, dout5 = _LAYER_DIMS[5]
    ig5, og5 = _CFG[5]
    g5 = np.minimum(np.arange(dout5) // (dout5 // og5), og5 - 1)
    s5 = np.zeros((ig5 * og5, ig5 * 128), np.float32)
    for i in range(ig5):
        s5[i * og5 + g5, i * 128 + np.arange(dout5)] = 1.0
    return sin, bmask, s5


_SIN, _BMASK, _S5 = _build_selectors()


def _dot11(a, b):
    return jax.lax.dot_general(
        a, b, (((1,), (1,)), ((), ())), preferred_element_type=jnp.float32)


def _dot10(a, b):
    return jax.lax.dot_general(
        a, b, (((1,), (0,)), ((), ())), preferred_element_type=jnp.float32)


def _body(x_ref, *refs):
    w_refs = refs[0:6]
    b_refs = refs[6:12]
    rw_refs = refs[12:18]
    rb_refs = refs[18:24]
    sin_refs = refs[24:29]
    bm_refs = refs[29:35]
    s5_ref = refs[35]
    o_ref = refs[36]

    cur = x_ref[...]
    for li in range(6):
        din, dout = _LAYER_DIMS[li]
        ig, og = _CFG[li]
        w = w_refs[li][...]
        b = b_refs[li][...]

        scores = _dot11(cur, rw_refs[li][...]) + rb_refs[li][...]
        m = jnp.max(scores, axis=-1, keepdims=True)
        e = jnp.exp(scores - m)
        pw = e / jnp.sum(e, axis=-1, keepdims=True)

        bias_out = _dot10(pw, bm_refs[li][...] * b)

        if li < 5:
            dpad = _pad128(din)
            wo = dout // og
            gexp = _dot10(pw, sin_refs[li][...])
            outs = []
            for j in range(og):
                gated = cur * gexp[:, j * dpad:j * dpad + din]
                outs.append(_dot11(gated, w[j * wo:(j + 1) * wo, :]))
            out = jnp.concatenate(outs, axis=1) + bias_out
            out = 0.5 * out * (1.0 + jax.lax.erf(out * 0.7071067811865476))
        else:
            wi = din // ig
            gall = _dot10(pw, s5_ref[...])
            out = bias_out
            for i in range(ig):
                a = _dot11(cur[:, i * wi:(i + 1) * wi], w[:, i * wi:(i + 1) * wi])
                out = out + a * gall[:, i * 128:i * 128 + dout]
        cur = out

    o_ref[...] = cur


def kernel(x, fc_w, fc_b, rt_w, rt_b):
    batch = x.shape[0]
    bb = _BB if batch % _BB == 0 else batch

    w_list = [w.at[0, :].set(0.0) for w in fc_w]
    w_list[0] = w_list[0].at[:, 0].set(0.0)
    b_list = [jnp.reshape(v.at[0].set(0.0), (1, -1)) for v in fc_b]
    rb_list = [jnp.reshape(v, (1, -1)) for v in rt_b]

    full = lambda arr: pl.BlockSpec(arr.shape, lambda i: (0, 0))
    in_specs = [pl.BlockSpec((bb, x.shape[1]), lambda i: (i, 0))]
    operands = [x]
    for group in (w_list, b_list, list(rt_w), rb_list,
                  [jnp.asarray(s) for s in _SIN],
                  [jnp.asarray(s) for s in _BMASK],
                  [jnp.asarray(_S5)]):
        for arr in group:
            in_specs.append(full(arr))
            operands.append(arr)

    return pl.pallas_call(
        _body,
        grid=(batch // bb,),
        in_specs=in_specs,
        out_specs=pl.BlockSpec((bb, 10), lambda i: (i, 0)),
        out_shape=jax.ShapeDtypeStruct((batch, 10), jnp.float32),
        compiler_params=pltpu.CompilerParams(
            dimension_semantics=("parallel",)),
    )(*operands)


# speedup vs baseline: 1.0041x; 1.0041x over previous
import numpy as np

import jax
import jax.numpy as jnp
from jax.experimental import pallas as pl
from jax.experimental.pallas import tpu as pltpu

_LAYER_DIMS = [(784, 512), (512, 256), (256, 128), (128, 64), (64, 32), (32, 10)]
_CFG = [(4, 2), (2, 2), (2, 2), (2, 2), (2, 2), (2, 4)]

_BB = 2048
_NH = 2


def _dot11(a, b):
    return jax.lax.dot_general(
        a, b, (((1,), (1,)), ((), ())), preferred_element_type=jnp.float32)


def _layer(cur, li, w, b, rw, rb):
    din, dout = _LAYER_DIMS[li]
    ig, og = _CFG[li]
    wi = din // ig
    wo = [dout // og] * og
    wo[-1] = dout - (og - 1) * (dout // og)
    woff = np.cumsum([0] + wo)

    scores = _dot11(cur, rw) + rb
    m = jnp.max(scores, axis=-1, keepdims=True)
    e = jnp.exp(scores - m)
    pw = e / jnp.sum(e, axis=-1, keepdims=True)

    parts = [_dot11(cur[:, i * wi:(i + 1) * wi], w[:, i * wi:(i + 1) * wi]) + b
             for i in range(ig)]
    outs = []
    for j in range(og):
        acc = None
        for i in range(ig):
            t = parts[i][:, woff[j]:woff[j + 1]] * pw[:, i * og + j:i * og + j + 1]
            acc = t if acc is None else acc + t
        outs.append(acc)
    out = jnp.concatenate(outs, axis=1)
    if li < 5:
        out = 0.5 * out * (1.0 + jax.lax.erf(out * 0.7071067811865476))
    return out


def _body(x_ref, *refs):
    w_refs = refs[0:6]
    b_refs = refs[6:12]
    rw_refs = refs[12:18]
    rb_refs = refs[18:24]
    o_ref = refs[24]

    hb = x_ref.shape[0] // _NH
    curs = [x_ref[h * hb:(h + 1) * hb] for h in range(_NH)]
    for li in range(6):
        w = w_refs[li][...]
        b = b_refs[li][...]
        rw = rw_refs[li][...]
        rb = rb_refs[li][...]
        curs = [_layer(curs[h], li, w, b, rw, rb) for h in range(_NH)]
    for h in range(_NH):
        o_ref[h * hb:(h + 1) * hb, :] = curs[h]


def kernel(x, fc_w, fc_b, rt_w, rt_b):
    batch = x.shape[0]
    bb = _BB if batch % _BB == 0 else batch

    w_list = [w.at[0, :].set(0.0) for w in fc_w]
    w_list[0] = w_list[0].at[:, 0].set(0.0)
    b_list = [jnp.reshape(v.at[0].set(0.0), (1, -1)) for v in fc_b]
    rb_list = [jnp.reshape(v, (1, -1)) for v in rt_b]

    full = lambda arr: pl.BlockSpec(arr.shape, lambda i: (0, 0))
    in_specs = [pl.BlockSpec((bb, x.shape[1]), lambda i: (i, 0))]
    operands = [x]
    for group in (w_list, b_list, list(rt_w), rb_list):
        for arr in group:
            in_specs.append(full(arr))
            operands.append(arr)

    return pl.pallas_call(
        _body,
        grid=(batch // bb,),
        in_specs=in_specs,
        out_specs=pl.BlockSpec((bb, 10), lambda i: (i, 0)),
        out_shape=jax.ShapeDtypeStruct((batch, 10), jnp.float32),
        compiler_params=pltpu.CompilerParams(
            dimension_semantics=("parallel",)),
    )(*operands)


# speedup vs baseline: 1.0558x; 1.0515x over previous
import numpy as np

import jax
import jax.numpy as jnp
from jax.experimental import pallas as pl
from jax.experimental.pallas import tpu as pltpu

_LAYER_DIMS = [(784, 512), (512, 256), (256, 128), (128, 64), (64, 32), (32, 10)]
_CFG = [(4, 2), (2, 2), (2, 2), (2, 2), (2, 2), (2, 4)]

_BB = 2048
_NH = 1


def _dot11(a, b):
    return jax.lax.dot_general(
        a, b, (((1,), (1,)), ((), ())), preferred_element_type=jnp.float32)


def _layer(cur, li, w, b, rw, rb):
    din, dout = _LAYER_DIMS[li]
    ig, og = _CFG[li]
    wi = din // ig
    wo = [dout // og] * og
    wo[-1] = dout - (og - 1) * (dout // og)
    woff = np.cumsum([0] + wo)

    scores = _dot11(cur, rw) + rb
    m = jnp.max(scores, axis=-1, keepdims=True)
    e = jnp.exp(scores - m)
    pw = e / jnp.sum(e, axis=-1, keepdims=True)

    parts = [_dot11(cur[:, i * wi:(i + 1) * wi], w[:, i * wi:(i + 1) * wi]) + b
             for i in range(ig)]
    outs = []
    for j in range(og):
        acc = None
        for i in range(ig):
            t = parts[i][:, woff[j]:woff[j + 1]] * pw[:, i * og + j:i * og + j + 1]
            acc = t if acc is None else acc + t
        outs.append(acc)
    out = jnp.concatenate(outs, axis=1)
    if li < 5:
        out = 0.5 * out * (1.0 + jax.lax.erf(out * 0.7071067811865476))
    return out


def _body(x_ref, *refs):
    w_refs = refs[0:6]
    b_refs = refs[6:12]
    rw_refs = refs[12:18]
    rb_refs = refs[18:24]
    o_ref = refs[24]

    hb = x_ref.shape[0] // _NH
    curs = [x_ref[h * hb:(h + 1) * hb] for h in range(_NH)]
    for li in range(6):
        w = w_refs[li][...]
        b = b_refs[li][...]
        rw = rw_refs[li][...]
        rb = rb_refs[li][...]
        curs = [_layer(curs[h], li, w, b, rw, rb) for h in range(_NH)]
    for h in range(_NH):
        o_ref[h * hb:(h + 1) * hb, :] = curs[h]


def kernel(x, fc_w, fc_b, rt_w, rt_b):
    batch = x.shape[0]
    bb = _BB if batch % _BB == 0 else batch

    w_list = [w.at[0, :].set(0.0) for w in fc_w]
    w_list[0] = w_list[0].at[:, 0].set(0.0)
    b_list = [jnp.reshape(v.at[0].set(0.0), (1, -1)) for v in fc_b]
    rb_list = [jnp.reshape(v, (1, -1)) for v in rt_b]

    full = lambda arr: pl.BlockSpec(arr.shape, lambda i: (0, 0))
    in_specs = [pl.BlockSpec((bb, x.shape[1]), lambda i: (i, 0))]
    operands = [x]
    for group in (w_list, b_list, list(rt_w), rb_list):
        for arr in group:
            in_specs.append(full(arr))
            operands.append(arr)

    return pl.pallas_call(
        _body,
        grid=(batch // bb,),
        in_specs=in_specs,
        out_specs=pl.BlockSpec((bb, 10), lambda i: (i, 0)),
        out_shape=jax.ShapeDtypeStruct((batch, 10), jnp.float32),
        compiler_params=pltpu.CompilerParams(
            dimension_semantics=("parallel",)),
    )(*operands)
